# baseline (speedup 1.0000x reference)
import functools

import jax
import jax.numpy as jnp
from jax import lax
from jax.experimental import pallas as pl
from jax.experimental.pallas import tpu as pltpu
from jax.experimental.pallas import tpu_sc as plsc

VOCAB = 70873
EMBED_DIM = 128
BATCH = 4096
SEQ = 50

NC = 2
NS = 16
NW = NC * NS
B = BATCH * SEQ
BPW = B // NW
CHUNK = 128
NCH = BPW // CHUNK
NBUF = 7
NMAIN = NCH - (NCH % NBUF)


def _emb_body(ids_hbm, table_hbm, out_hbm, idx_v, rows_v, gsem, ssem):
    wid = lax.axis_index("s") * NC + lax.axis_index("c")
    base = wid * BPW

    pltpu.sync_copy(ids_hbm.at[wid], idx_v)

    def start_gather(j, b):
        pltpu.async_copy(table_hbm.at[idx_v.at[j]], rows_v.at[b], gsem)

    def start_store(j, b):
        pltpu.async_copy(
            rows_v.at[b], out_hbm.at[pl.ds(base + j * CHUNK, CHUNK)], ssem
        )

    def wait_gather(b):
        pltpu.make_async_copy(
            table_hbm.at[idx_v.at[0]], rows_v.at[b], gsem
        ).wait()

    def wait_store(b):
        pltpu.make_async_copy(rows_v.at[b], out_hbm.at[pl.ds(0, CHUNK)], ssem).wait()

    for b in range(NBUF):
        start_gather(b, b)

    @pl.loop(0, NMAIN, step=NBUF)
    def _(j0):
        for db in range(NBUF):
            j = j0 + db
            wait_gather(db)
            start_store(j, db)
            wait_store(db)

            @pl.when(j + NBUF < NCH)
            def _():
                start_gather(j + NBUF, db)

    for r in range(NMAIN, NCH):
        wait_gather(r % NBUF)
        start_store(r, r % NBUF)
        wait_store(r % NBUF)


@functools.cache
def _build():
    mesh = plsc.VectorSubcoreMesh(core_axis_name="c", subcore_axis_name="s")
    return functools.partial(
        pl.kernel,
        mesh=mesh,
        out_type=jax.ShapeDtypeStruct((B, EMBED_DIM), jnp.float32),
        scratch_types=[
            pltpu.VMEM((NCH, CHUNK), jnp.int32),
            pltpu.VMEM((NBUF, CHUNK, EMBED_DIM), jnp.float32),
            pltpu.SemaphoreType.DMA,
            pltpu.SemaphoreType.DMA,
        ],
    )(_emb_body)


def kernel(input_ids, table):
    ids = input_ids.T.reshape(NW, NCH, CHUNK).astype(jnp.int32)
    out = _build()(ids, table)
    return out.reshape(SEQ, BATCH, EMBED_DIM).transpose(1, 0, 2)


# speedup vs baseline: 1.0009x; 1.0009x over previous
import functools

import jax
import jax.numpy as jnp
from jax import lax
from jax.experimental import pallas as pl
from jax.experimental.pallas import tpu as pltpu
from jax.experimental.pallas import tpu_sc as plsc

VOCAB = 70873
EMBED_DIM = 128
BATCH = 4096
SEQ = 50

NC = 2
NS = 16
NW = NC * NS
B = BATCH * SEQ
BPW = B // NW
CHUNK = 128
NCH = BPW // CHUNK
NBUF = 5


def _emb_body(ids_hbm, table_hbm, out_hbm, idx_v, rows_v, gsem, ssem):
    wid = lax.axis_index("s") * NC + lax.axis_index("c")
    base = wid * BPW

    pltpu.sync_copy(ids_hbm.at[wid], idx_v)

    def start_gather(j, b):
        pltpu.async_copy(table_hbm.at[idx_v.at[j]], rows_v.at[b], gsem)

    def start_store(j, b):
        pltpu.async_copy(
            rows_v.at[b], out_hbm.at[pl.ds(base + j * CHUNK, CHUNK)], ssem
        )

    def wait_gather(b):
        pltpu.make_async_copy(
            table_hbm.at[idx_v.at[0]], rows_v.at[b], gsem
        ).wait()

    def wait_store(b):
        pltpu.make_async_copy(rows_v.at[b], out_hbm.at[pl.ds(0, CHUNK)], ssem).wait()

    for b in range(NBUF):
        start_gather(b, b)

    @pl.loop(0, NCH - NBUF, step=NBUF)
    def _(j0):
        for db in range(NBUF):
            j = j0 + db
            wait_gather(db)
            start_store(j, db)
            wait_store(db)
            start_gather(j + NBUF, db)

    for db in range(NBUF):
        wait_gather(db)
        start_store(NCH - NBUF + db, db)
    for db in range(NBUF):
        wait_store(db)


@functools.cache
def _build():
    mesh = plsc.VectorSubcoreMesh(core_axis_name="c", subcore_axis_name="s")
    return functools.partial(
        pl.kernel,
        mesh=mesh,
        out_type=jax.ShapeDtypeStruct((B, EMBED_DIM), jnp.float32),
        scratch_types=[
            pltpu.VMEM((NCH, CHUNK), jnp.int32),
            pltpu.VMEM((NBUF, CHUNK, EMBED_DIM), jnp.float32),
            pltpu.SemaphoreType.DMA,
            pltpu.SemaphoreType.DMA,
        ],
    )(_emb_body)


def kernel(input_ids, table):
    ids = input_ids.T.reshape(NW, NCH, CHUNK).astype(jnp.int32)
    out = _build()(ids, table)
    return out.reshape(SEQ, BATCH, EMBED_DIM).transpose(1, 0, 2)


# speedup vs baseline: 1.0116x; 1.0107x over previous
import functools

import jax
import jax.numpy as jnp
from jax import lax
from jax.experimental import pallas as pl
from jax.experimental.pallas import tpu as pltpu
from jax.experimental.pallas import tpu_sc as plsc

VOCAB = 70873
EMBED_DIM = 128
BATCH = 4096
SEQ = 50

NC = 2
NS = 16
NW = NC * NS
B = BATCH * SEQ
BPW = B // NW
CHUNK = 128
NCH = BPW // CHUNK
NBUF = 5
DELTA = 2


def _emb_body(ids_hbm, table_hbm, out_hbm, idx_v, rows_v, gsem, ssem):
    wid = lax.axis_index("s") * NC + lax.axis_index("c")
    base = wid * BPW

    pltpu.sync_copy(ids_hbm.at[wid], idx_v)

    def start_gather(j, b):
        pltpu.async_copy(table_hbm.at[idx_v.at[j]], rows_v.at[b], gsem)

    def start_store(j, b):
        pltpu.async_copy(
            rows_v.at[b], out_hbm.at[pl.ds(base + j * CHUNK, CHUNK)], ssem
        )

    def wait_gather(b):
        pltpu.make_async_copy(
            table_hbm.at[idx_v.at[0]], rows_v.at[b], gsem
        ).wait()

    def wait_store(b):
        pltpu.make_async_copy(rows_v.at[b], out_hbm.at[pl.ds(0, CHUNK)], ssem).wait()

    for b in range(NBUF):
        start_gather(b, b)

    @pl.loop(0, NCH, step=NBUF)
    def _(j0):
        for db in range(NBUF):
            j = j0 + db
            wait_gather(db)
            start_store(j, db)

            @pl.when(jnp.logical_and(j >= DELTA, j < NCH - NBUF + DELTA))
            def _():
                wait_store(db)
                start_gather(j + NBUF - DELTA, (db + NBUF - DELTA) % NBUF)

    for db in range(NBUF):
        wait_store(db)


@functools.cache
def _build():
    mesh = plsc.VectorSubcoreMesh(core_axis_name="c", subcore_axis_name="s")
    return functools.partial(
        pl.kernel,
        mesh=mesh,
        out_type=jax.ShapeDtypeStruct((B, EMBED_DIM), jnp.float32),
        scratch_types=[
            pltpu.VMEM((NCH, CHUNK), jnp.int32),
            pltpu.VMEM((NBUF, CHUNK, EMBED_DIM), jnp.float32),
            pltpu.SemaphoreType.DMA,
            pltpu.SemaphoreType.DMA,
        ],
    )(_emb_body)


def kernel(input_ids, table):
    ids = input_ids.T.reshape(NW, NCH, CHUNK).astype(jnp.int32)
    out = _build()(ids, table)
    return out.reshape(SEQ, BATCH, EMBED_DIM).transpose(1, 0, 2)
